# baseline (speedup 1.0000x reference)
import jax
import jax.numpy as jnp
from jax import lax
from jax.experimental import pallas as pl
from jax.experimental.pallas import tpu as pltpu

_NB_VEC = 8
_HC = 8


def _rv_kernel(x_ref, vec_ref, sim_ref, pos_ref):
    C, H, W = x_ref.shape[1:]
    nslab = H // _HC
    flat = (lax.broadcasted_iota(jnp.int32, (H, W), 0) * W
            + lax.broadcasted_iota(jnp.int32, (H, W), 1))
    lane = lax.broadcasted_iota(jnp.int32, (1, 1, W), 2)
    selpos = jnp.zeros((H, W), jnp.float32)
    score = jnp.zeros((H, W), jnp.float32)
    xn2 = None
    sim_prev = None
    ssum_prev = None
    for i in range(_NB_VEC):
        if i == 0:
            n = jnp.int32((H * W) // 2)
        else:
            m = jnp.max(score)
            n = jnp.min(jnp.where(score >= m, flat, jnp.int32(2 ** 30)))
        io = n // W
        jo = n % W
        selpos = selpos + (flat == n).astype(jnp.float32)
        row = x_ref[0, :, pl.ds(io, 1), :]
        raw = jnp.sum(row * (lane == jo).astype(jnp.float32),
                      axis=2, keepdims=True)
        rn2 = jnp.sum(raw * raw)
        dots_parts = []
        xn2_parts = []
        wacc = jnp.zeros((C, W), jnp.float32)
        for s in range(nslab):
            sl = slice(s * _HC, (s + 1) * _HC)
            Xs = x_ref[0, :, sl, :]
            dots_parts.append(jnp.sum(Xs * raw, axis=0))
            if i == 0:
                xn2_parts.append(jnp.sum(Xs * Xs, axis=0))
            else:
                wacc = wacc + jnp.sum(Xs * sim_prev[None, sl, :], axis=1)
        dots = jnp.concatenate(dots_parts, axis=0)
        if i == 0:
            xn2 = jnp.concatenate(xn2_parts, axis=0)
        else:
            vec_ref[0, i - 1, :] = jnp.sum(wacc, axis=1) * (1.0 / ssum_prev)
        d2 = jnp.maximum(xn2 - 2.0 * dots + rn2, 0.0)
        sim = jnp.exp(jnp.sqrt(d2) * (-1.0 / 20.0))
        sim_ref[0, i, :, :] = sim
        score = (1.0 - sim) if i == 0 else (1.0 - sim) * score
        sim_prev = sim
        ssum_prev = jnp.sum(sim)
    wacc = jnp.zeros((C, W), jnp.float32)
    for s in range(nslab):
        sl = slice(s * _HC, (s + 1) * _HC)
        Xs = x_ref[0, :, sl, :]
        wacc = wacc + jnp.sum(Xs * sim_prev[None, sl, :], axis=1)
    vec_ref[0, _NB_VEC - 1, :] = jnp.sum(wacc, axis=1) * (1.0 / ssum_prev)
    pos_ref[0, 0, :, :] = selpos


def kernel(x, applyUMAP):
    B, C, H, W = x.shape
    vecs, sims, selpos = pl.pallas_call(
        _rv_kernel,
        grid=(B,),
        in_specs=[pl.BlockSpec((1, C, H, W), lambda b: (b, 0, 0, 0))],
        out_specs=[
            pl.BlockSpec((1, _NB_VEC, C), lambda b: (b, 0, 0)),
            pl.BlockSpec((1, _NB_VEC, H, W), lambda b: (b, 0, 0, 0)),
            pl.BlockSpec((1, 1, H, W), lambda b: (b, 0, 0, 0)),
        ],
        out_shape=[
            jax.ShapeDtypeStruct((B, _NB_VEC, C), jnp.float32),
            jax.ShapeDtypeStruct((B, _NB_VEC, H, W), jnp.float32),
            jax.ShapeDtypeStruct((B, 1, H, W), jnp.float32),
        ],
        compiler_params=pltpu.CompilerParams(
            dimension_semantics=("parallel",),
        ),
    )(x)
    return vecs, sims, selpos


# speedup vs baseline: 2.3460x; 2.3460x over previous
import jax
import jax.numpy as jnp
from jax import lax
from jax.experimental import pallas as pl
from jax.experimental.pallas import tpu as pltpu

_NB_VEC = 8
_BB = 2


def _rv_kernel(x_ref, vec_ref, sim_ref, pos_ref):
    C, H, W = x_ref.shape[1:]
    flat = (lax.broadcasted_iota(jnp.int32, (H, W), 0) * W
            + lax.broadcasted_iota(jnp.int32, (H, W), 1))
    lane = lax.broadcasted_iota(jnp.int32, (1, 1, W), 2)
    for bb in range(_BB):
        X = x_ref[bb]
        xn2 = jnp.sum(X * X, axis=0)
        selpos = jnp.zeros((H, W), jnp.float32)
        score = jnp.zeros((H, W), jnp.float32)
        for i in range(_NB_VEC):
            if i == 0:
                n = jnp.int32((H * W) // 2)
            else:
                m = jnp.max(score)
                n = jnp.min(jnp.where(score >= m, flat, jnp.int32(2 ** 30)))
            io = n // W
            jo = n % W
            selpos = selpos + (flat == n).astype(jnp.float32)
            row = x_ref[bb, :, pl.ds(io, 1), :]
            raw = jnp.sum(row * (lane == jo).astype(jnp.float32),
                          axis=2, keepdims=True)
            rn2 = jnp.sum(raw * raw)
            dots = jnp.sum(X * raw, axis=0)
            d2 = jnp.maximum(xn2 - 2.0 * dots + rn2, 0.0)
            sim = jnp.exp(jnp.sqrt(d2) * (-1.0 / 20.0))
            wsum = jnp.sum(X * sim[None, :, :], axis=(1, 2))
            vec_ref[bb, i, :] = wsum * (1.0 / jnp.sum(sim))
            sim_ref[bb, i, :, :] = sim
            score = (1.0 - sim) if i == 0 else (1.0 - sim) * score
        pos_ref[bb, 0, :, :] = selpos


def kernel(x, applyUMAP):
    B, C, H, W = x.shape
    vecs, sims, selpos = pl.pallas_call(
        _rv_kernel,
        grid=(B // _BB,),
        in_specs=[pl.BlockSpec((_BB, C, H, W), lambda b: (b, 0, 0, 0))],
        out_specs=[
            pl.BlockSpec((_BB, _NB_VEC, C), lambda b: (b, 0, 0)),
            pl.BlockSpec((_BB, _NB_VEC, H, W), lambda b: (b, 0, 0, 0)),
            pl.BlockSpec((_BB, 1, H, W), lambda b: (b, 0, 0, 0)),
        ],
        out_shape=[
            jax.ShapeDtypeStruct((B, _NB_VEC, C), jnp.float32),
            jax.ShapeDtypeStruct((B, _NB_VEC, H, W), jnp.float32),
            jax.ShapeDtypeStruct((B, 1, H, W), jnp.float32),
        ],
        compiler_params=pltpu.CompilerParams(
            dimension_semantics=("parallel",),
        ),
    )(x)
    return vecs, sims, selpos


# speedup vs baseline: 2.4911x; 1.0619x over previous
import jax
import jax.numpy as jnp
from jax import lax
from jax.experimental import pallas as pl
from jax.experimental.pallas import tpu as pltpu

_NB_VEC = 8


def _rv_kernel(x_ref, vec_ref, sim_ref, pos_ref):
    X = x_ref[0]
    C, H, W = X.shape
    flat = (lax.broadcasted_iota(jnp.int32, (H, W), 0) * W
            + lax.broadcasted_iota(jnp.int32, (H, W), 1))
    lane = lax.broadcasted_iota(jnp.int32, (1, 1, W), 2)
    xn2 = jnp.sum(X * X, axis=0)
    selpos = jnp.zeros((H, W), jnp.float32)
    score = jnp.zeros((H, W), jnp.float32)

    def gather(n):
        io = n // W
        jo = n % W
        row = x_ref[0, :, pl.ds(io, 1), :]
        raw = jnp.sum(row * (lane == jo).astype(jnp.float32),
                      axis=2, keepdims=True)
        return raw

    n = jnp.int32((H * W) // 2)
    raw = gather(n)
    for i in range(_NB_VEC):
        selpos = selpos + (flat == n).astype(jnp.float32)
        rn2 = jnp.sum(raw * raw)
        dots = jnp.sum(X * raw, axis=0)
        d2 = jnp.maximum(xn2 - 2.0 * dots + rn2, 0.0)
        sim = jnp.exp(jnp.sqrt(d2) * (-1.0 / 20.0))
        sim_ref[0, i, :, :] = sim
        score = (1.0 - sim) if i == 0 else (1.0 - sim) * score
        if i + 1 < _NB_VEC:
            m = jnp.max(score)
            n = jnp.min(jnp.where(score >= m, flat, jnp.int32(2 ** 30)))
            raw = gather(n)
        wsum = jnp.sum(X * sim[None, :, :], axis=(1, 2))
        vec_ref[0, i, :] = wsum * (1.0 / jnp.sum(sim))
    pos_ref[0, 0, :, :] = selpos


def kernel(x, applyUMAP):
    B, C, H, W = x.shape
    vecs, sims, selpos = pl.pallas_call(
        _rv_kernel,
        grid=(B,),
        in_specs=[pl.BlockSpec((1, C, H, W), lambda b: (b, 0, 0, 0))],
        out_specs=[
            pl.BlockSpec((1, _NB_VEC, C), lambda b: (b, 0, 0)),
            pl.BlockSpec((1, _NB_VEC, H, W), lambda b: (b, 0, 0, 0)),
            pl.BlockSpec((1, 1, H, W), lambda b: (b, 0, 0, 0)),
        ],
        out_shape=[
            jax.ShapeDtypeStruct((B, _NB_VEC, C), jnp.float32),
            jax.ShapeDtypeStruct((B, _NB_VEC, H, W), jnp.float32),
            jax.ShapeDtypeStruct((B, 1, H, W), jnp.float32),
        ],
        compiler_params=pltpu.CompilerParams(
            dimension_semantics=("parallel",),
        ),
    )(x)
    return vecs, sims, selpos


# speedup vs baseline: 2.6233x; 1.0531x over previous
import jax
import jax.numpy as jnp
from jax import lax
from jax.experimental import pallas as pl
from jax.experimental.pallas import tpu as pltpu

_NB_VEC = 8


def _rv_kernel(x_ref, vec_ref, sim_ref, pos_ref):
    X = x_ref[0]
    C, H, W = X.shape
    flat = (lax.broadcasted_iota(jnp.int32, (H, W), 0) * W
            + lax.broadcasted_iota(jnp.int32, (H, W), 1))
    lane = lax.broadcasted_iota(jnp.int32, (1, 1, W), 2)
    xn2 = jnp.sum(X * X, axis=0)
    selpos = jnp.zeros((H, W), jnp.float32)
    score = jnp.zeros((H, W), jnp.float32)

    def gather(n):
        io = n // W
        jo = n % W
        row = x_ref[0, :, pl.ds(io, 1), :]
        raw = pltpu.roll(row, -jo, axis=2)[:, :, 0:1]
        return raw

    n = jnp.int32((H * W) // 2)
    raw = gather(n)
    for i in range(_NB_VEC):
        selpos = selpos + (flat == n).astype(jnp.float32)
        rn2 = jnp.sum(raw * raw)
        dots = jnp.sum(X * raw, axis=0)
        d2 = jnp.maximum(xn2 - 2.0 * dots + rn2, 0.0)
        sim = jnp.exp(jnp.sqrt(d2) * (-1.0 / 20.0))
        sim_ref[0, i, :, :] = sim
        score = (1.0 - sim) if i == 0 else (1.0 - sim) * score
        if i + 1 < _NB_VEC:
            m = jnp.max(score)
            n = jnp.min(jnp.where(score >= m, flat, jnp.int32(2 ** 30)))
            raw = gather(n)
        wsum = jnp.sum(X * sim[None, :, :], axis=(1, 2))
        vec_ref[0, i, :] = wsum * (1.0 / jnp.sum(sim))
    pos_ref[0, 0, :, :] = selpos


def kernel(x, applyUMAP):
    B, C, H, W = x.shape
    vecs, sims, selpos = pl.pallas_call(
        _rv_kernel,
        grid=(B,),
        in_specs=[pl.BlockSpec((1, C, H, W), lambda b: (b, 0, 0, 0))],
        out_specs=[
            pl.BlockSpec((1, _NB_VEC, C), lambda b: (b, 0, 0)),
            pl.BlockSpec((1, _NB_VEC, H, W), lambda b: (b, 0, 0, 0)),
            pl.BlockSpec((1, 1, H, W), lambda b: (b, 0, 0, 0)),
        ],
        out_shape=[
            jax.ShapeDtypeStruct((B, _NB_VEC, C), jnp.float32),
            jax.ShapeDtypeStruct((B, _NB_VEC, H, W), jnp.float32),
            jax.ShapeDtypeStruct((B, 1, H, W), jnp.float32),
        ],
        compiler_params=pltpu.CompilerParams(
            dimension_semantics=("parallel",),
        ),
    )(x)
    return vecs, sims, selpos


# speedup vs baseline: 2.6331x; 1.0038x over previous
import jax
import jax.numpy as jnp
from jax import lax
from jax.experimental import pallas as pl
from jax.experimental.pallas import tpu as pltpu

_NB_VEC = 8


def _rv_kernel(x_ref, vec_ref, sim_ref, pos_ref):
    X = x_ref[0]
    C, H, W = X.shape
    flat = (lax.broadcasted_iota(jnp.int32, (H, W), 0) * W
            + lax.broadcasted_iota(jnp.int32, (H, W), 1))
    lane = lax.broadcasted_iota(jnp.int32, (1, 1, W), 2)
    xn2 = jnp.sum(X * X, axis=0)
    pos_ref[0, 0, :, :] = jnp.zeros((H, W), jnp.float32)
    score = jnp.zeros((H, W), jnp.float32)

    def gather(n):
        io = n // W
        jo = n % W
        row = x_ref[0, :, pl.ds(io, 1), :]
        raw = pltpu.roll(row, -jo, axis=2)[:, :, 0:1]
        return raw

    n = jnp.int32((H * W) // 2)
    raw = gather(n)
    for i in range(_NB_VEC):
        io = n // W
        jo = n % W
        prow = pos_ref[0, 0, pl.ds(io, 1), :]
        pos_ref[0, 0, pl.ds(io, 1), :] = (
            prow + (lane[0] == jo).astype(jnp.float32))
        rn2 = jnp.sum(raw * raw)
        dots = jnp.sum(X * raw, axis=0)
        d2 = jnp.maximum(xn2 - 2.0 * dots + rn2, 0.0)
        sim = jnp.exp(jnp.sqrt(d2) * (-1.0 / 20.0))
        sim_ref[0, i, :, :] = sim
        score = (1.0 - sim) if i == 0 else (1.0 - sim) * score
        if i + 1 < _NB_VEC:
            m = jnp.max(score)
            n = jnp.min(jnp.where(score >= m, flat, jnp.int32(2 ** 30)))
            raw = gather(n)
        wsum = jnp.sum(X * sim[None, :, :], axis=(1, 2))
        vec_ref[0, i, :] = wsum * (1.0 / jnp.sum(sim))


def kernel(x, applyUMAP):
    B, C, H, W = x.shape
    vecs, sims, selpos = pl.pallas_call(
        _rv_kernel,
        grid=(B,),
        in_specs=[pl.BlockSpec((1, C, H, W), lambda b: (b, 0, 0, 0))],
        out_specs=[
            pl.BlockSpec((1, _NB_VEC, C), lambda b: (b, 0, 0)),
            pl.BlockSpec((1, _NB_VEC, H, W), lambda b: (b, 0, 0, 0)),
            pl.BlockSpec((1, 1, H, W), lambda b: (b, 0, 0, 0)),
        ],
        out_shape=[
            jax.ShapeDtypeStruct((B, _NB_VEC, C), jnp.float32),
            jax.ShapeDtypeStruct((B, _NB_VEC, H, W), jnp.float32),
            jax.ShapeDtypeStruct((B, 1, H, W), jnp.float32),
        ],
        compiler_params=pltpu.CompilerParams(
            dimension_semantics=("parallel",),
        ),
    )(x)
    return vecs, sims, selpos


# speedup vs baseline: 2.6350x; 1.0007x over previous
import jax
import jax.numpy as jnp
from jax import lax
from jax.experimental import pallas as pl
from jax.experimental.pallas import tpu as pltpu

_NB_VEC = 8


def _rv_kernel(x_ref, vec_ref, sim_ref, pos_ref):
    X = x_ref[0]
    C, H, W = X.shape
    flat = (lax.broadcasted_iota(jnp.int32, (H, W), 0) * W
            + lax.broadcasted_iota(jnp.int32, (H, W), 1))
    lane = lax.broadcasted_iota(jnp.int32, (1, 1, W), 2)
    xn2 = jnp.sum(X * X, axis=0)
    pos_ref[0, 0, :, :] = jnp.zeros((H, W), jnp.float32)
    score = jnp.zeros((H, W), jnp.float32)

    def gather(n):
        io = n // W
        jo = n % W
        row = x_ref[0, :, pl.ds(io, 1), :]
        raw = pltpu.roll(row, -jo, axis=2)[:, :, 0:1]
        return raw

    n = jnp.int32((H * W) // 2)
    raw = gather(n)
    for i in range(_NB_VEC):
        io = n // W
        jo = n % W
        prow = pos_ref[0, 0, pl.ds(io, 1), :]
        pos_ref[0, 0, pl.ds(io, 1), :] = (
            prow + (lane[0] == jo).astype(jnp.float32))
        rn2 = jnp.sum(raw * raw)
        dots = jnp.sum(X * raw, axis=0)
        d2 = jnp.maximum(xn2 - 2.0 * dots + rn2, 0.0)
        sim = jnp.exp(jnp.sqrt(d2) * (-1.0 / 20.0))
        sim_ref[0, i, :, :] = sim
        score = (1.0 - sim) if i == 0 else (1.0 - sim) * score
        score = jnp.where(flat == n, 0.0, score)
        if i + 1 < _NB_VEC:
            m = jnp.max(score)
            n = jnp.min(jnp.where(score >= m, flat, jnp.int32(2 ** 30)))
            raw = gather(n)
        wsum = jnp.sum(X * sim[None, :, :], axis=(1, 2))
        vec_ref[0, i, :] = wsum * (1.0 / jnp.sum(sim))


def kernel(x, applyUMAP):
    B, C, H, W = x.shape
    vecs, sims, selpos = pl.pallas_call(
        _rv_kernel,
        grid=(B,),
        in_specs=[pl.BlockSpec((1, C, H, W), lambda b: (b, 0, 0, 0))],
        out_specs=[
            pl.BlockSpec((1, _NB_VEC, C), lambda b: (b, 0, 0)),
            pl.BlockSpec((1, _NB_VEC, H, W), lambda b: (b, 0, 0, 0)),
            pl.BlockSpec((1, 1, H, W), lambda b: (b, 0, 0, 0)),
        ],
        out_shape=[
            jax.ShapeDtypeStruct((B, _NB_VEC, C), jnp.float32),
            jax.ShapeDtypeStruct((B, _NB_VEC, H, W), jnp.float32),
            jax.ShapeDtypeStruct((B, 1, H, W), jnp.float32),
        ],
        compiler_params=pltpu.CompilerParams(
            dimension_semantics=("parallel",),
        ),
    )(x)
    return vecs, sims, selpos
